# baseline (speedup 1.0000x reference)
import functools

import jax
import jax.numpy as jnp
from jax.experimental import pallas as pl
from jax.experimental.pallas import tpu as pltpu

N = 10000
G = 32
C = 256
NOFF = 27


def _matmul_body(f_ref, w_ref, y_ref):
    y_ref[...] = jnp.dot(f_ref[...], w_ref[...],
                         preferred_element_type=jnp.float32)


def _big_matmul(f_pad, w_cat, n_pad):
    bm, bn = 2048, 768
    grid = (n_pad // bm, (NOFF * C) // bn)
    return pl.pallas_call(
        _matmul_body,
        grid=grid,
        in_specs=[
            pl.BlockSpec((bm, C), lambda i, j: (i, 0)),
            pl.BlockSpec((C, bn), lambda i, j: (0, j)),
        ],
        out_specs=pl.BlockSpec((bm, bn), lambda i, j: (i, j)),
        out_shape=jax.ShapeDtypeStruct((n_pad, NOFF * C), jnp.float32),
    )(f_pad, w_cat)


def kernel(features, inp_positions, W, voxel_size=1.0):
    Gp = G + 2
    v = jnp.floor(inp_positions / voxel_size).astype(jnp.int32)
    lin = (v[:, 0] + 1) * (Gp * Gp) + (v[:, 1] + 1) * Gp + (v[:, 2] + 1)
    table = jnp.full((Gp * Gp * Gp,), -1, dtype=jnp.int32).at[lin].set(
        jnp.arange(N, dtype=jnp.int32))

    n_pad = 10240
    f_pad = jnp.zeros((n_pad, C), jnp.float32).at[:N].set(features)
    w_cat = W.reshape(NOFF, C, C).transpose(1, 0, 2).reshape(C, NOFF * C)
    y = _big_matmul(f_pad, w_cat, n_pad)[:N].reshape(N, NOFF, C)

    offs = jnp.array(
        [dx * (Gp * Gp) + dy * Gp + dz
         for dx in (-1, 0, 1) for dy in (-1, 0, 1) for dz in (-1, 0, 1)],
        dtype=jnp.int32)
    nlin = lin[:, None] + offs[None, :]
    nidx = table[nlin]
    valid = nidx >= 0
    safe = jnp.clip(nidx, 0, N - 1)
    gathered = y[safe, jnp.arange(NOFF)[None, :], :]
    out = jnp.sum(jnp.where(valid[:, :, None], gathered, 0.0), axis=1)
    return out
